# Initial kernel scaffold; baseline (speedup 1.0000x reference)
import jax
import jax.numpy as jnp
from jax.experimental import pallas as pl


def kernel(x, y, emb_period, emb_stations, emb_platforms, w_period, w_stations, w_platforms, w_n, w_y, b_y):
    raise NotImplementedError("write your pallas kernel here")


# speedup vs baseline: 7.1602x; 7.1602x over previous
import functools

import jax
import jax.numpy as jnp
from jax import lax
from jax.experimental import pallas as pl
from jax.experimental.pallas import tpu as pltpu
from jax.experimental.pallas import tpu_sc as plsc

_N = 16384
_NC = 2
_NS = 16
_NW = _NC * _NS
_RPW = _N // _NW
_L = 16
_CHUNKS = _RPW // _L

_SP = 176
_LP = 464


def _sc_body(x_hbm, y_hbm, ep_hbm, es_hbm, el_hbm, wv_hbm, out_hbm,
             xb, yv, ov, epv, esv, elv, wvv, tP, tSa, tSb, tSc, tL):
    wid = lax.axis_index("s") * _NC + lax.axis_index("c")
    base = wid * _RPW

    pltpu.sync_copy(x_hbm.at[pl.ds(base * 12, _RPW * 12)], xb)
    pltpu.sync_copy(y_hbm.at[pl.ds(base, _RPW)], yv)
    pltpu.sync_copy(ep_hbm, epv)
    pltpu.sync_copy(es_hbm, esv)
    pltpu.sync_copy(el_hbm, elv)
    pltpu.sync_copy(wv_hbm, wvv)

    wlo = wvv[pl.ds(0, _L)]
    whi = wvv[pl.ds(8, _L)]

    def ws(k):
        return wlo[k] if k < _L else whi[k - 8]

    tP[...] = epv[0, :] * ws(0) + epv[1, :] * ws(1)
    for j in range(_SP // _L):
        sl = pl.ds(j * _L, _L)
        e0, e1, e2 = esv[0, sl], esv[1, sl], esv[2, sl]
        tSa[sl] = e0 * ws(2) + e1 * ws(3) + e2 * ws(4)
        tSb[sl] = e0 * ws(5) + e1 * ws(6) + e2 * ws(7)
        tSc[sl] = e0 * ws(8) + e1 * ws(9) + e2 * ws(10)
    for j in range(_LP // _L):
        sl = pl.ds(j * _L, _L)
        tL[sl] = (elv[0, sl] * ws(11) + elv[1, sl] * ws(12)
                  + elv[2, sl] * ws(13))

    lanes = lax.iota(jnp.int32, _L)
    for j in range(_CHUNKS):
        rbase = (lanes + (j * _L)) * 12
        sl = pl.ds(j * _L, _L)

        def col(c):
            return plsc.load_gather(xb, [rbase + c])

        acc = plsc.load_gather(tP, [col(0).astype(jnp.int32)])
        acc = acc + plsc.load_gather(tSa, [col(1).astype(jnp.int32)])
        acc = acc + plsc.load_gather(tSb, [col(2).astype(jnp.int32)])
        acc = acc + plsc.load_gather(tSc, [col(3).astype(jnp.int32)])
        acc = acc + plsc.load_gather(tL, [col(4).astype(jnp.int32)])
        for c in range(7):
            acc = acc + col(5 + c) * ws(14 + c)
        acc = acc + yv[sl] * ws(21) + ws(22)
        ov[sl] = acc

    pltpu.sync_copy(ov, out_hbm.at[pl.ds(base, _RPW)])


@jax.jit
def _run(x, yf, eP, eS, eL, wv):
    mesh = plsc.VectorSubcoreMesh(core_axis_name="c", subcore_axis_name="s",
                                  num_cores=_NC, num_subcores=_NS)
    f = pl.kernel(
        _sc_body,
        out_type=jax.ShapeDtypeStruct((_N,), jnp.float32),
        mesh=mesh,
        compiler_params=pltpu.CompilerParams(needs_layout_passes=False),
        scratch_types=[
            pltpu.VMEM((_RPW * 12,), jnp.float32),
            pltpu.VMEM((_RPW,), jnp.float32),
            pltpu.VMEM((_RPW,), jnp.float32),
            pltpu.VMEM((2, _L), jnp.float32),
            pltpu.VMEM((3, _SP), jnp.float32),
            pltpu.VMEM((3, _LP), jnp.float32),
            pltpu.VMEM((24,), jnp.float32),
            pltpu.VMEM((_L,), jnp.float32),
            pltpu.VMEM((_SP,), jnp.float32),
            pltpu.VMEM((_SP,), jnp.float32),
            pltpu.VMEM((_SP,), jnp.float32),
            pltpu.VMEM((_LP,), jnp.float32),
        ],
    )
    return f(x, yf, eP, eS, eL, wv)


def kernel(x, y, emb_period, emb_stations, emb_platforms, w_period,
           w_stations, w_platforms, w_n, w_y, b_y):
    if x.ndim == 1:
        x = x.reshape(1, -1)
    eP = jnp.zeros((2, _L), jnp.float32).at[:, :4].set(emb_period.T)
    eS = jnp.zeros((3, _SP), jnp.float32).at[:, :162].set(emb_stations.T)
    eL = jnp.zeros((3, _LP), jnp.float32).at[:, :460].set(emb_platforms.T)
    wv = jnp.concatenate([
        w_period[0], w_stations[0], w_platforms[0], w_n[0], w_y[0], b_y,
        jnp.zeros((1,), jnp.float32),
    ])
    out = _run(x.reshape(-1), y.reshape(-1), eP, eS, eL, wv)
    return out.reshape(-1, 1)


# speedup vs baseline: 8.4094x; 1.1745x over previous
import jax
import jax.numpy as jnp
from jax import lax
from jax.experimental import pallas as pl
from jax.experimental.pallas import tpu as pltpu
from jax.experimental.pallas import tpu_sc as plsc

_N = 16384
_NC = 2
_NS = 16
_NW = _NC * _NS
_RPW = _N // _NW
_L = 16
_CHUNKS = _RPW // _L

_SP = 176
_LP = 464


def _sc_body(x_hbm, y_hbm, ep_hbm, es_hbm, el_hbm, wp_hbm, wst_hbm,
             wpl_hbm, wn_hbm, wy_hbm, by_hbm, out_hbm,
             xb, yv, ov, epv, esv, elv, wv, tP, tSa, tSb, tSc, tL,
             sem_small, sem_big):
    wid = lax.axis_index("s") * _NC + lax.axis_index("c")
    base = wid * _RPW

    big = [
        pltpu.async_copy(x_hbm.at[pl.ds(base * 12, _RPW * 12)], xb, sem_big),
        pltpu.async_copy(y_hbm.at[pl.ds(base, _RPW)], yv, sem_big),
    ]
    small = [
        pltpu.async_copy(ep_hbm, epv.at[pl.ds(0, 8)], sem_small),
        pltpu.async_copy(es_hbm, esv.at[pl.ds(0, 486)], sem_small),
        pltpu.async_copy(el_hbm, elv.at[pl.ds(0, 1380)], sem_small),
        pltpu.async_copy(wp_hbm, wv.at[pl.ds(0, 2)], sem_small),
        pltpu.async_copy(wst_hbm, wv.at[pl.ds(16, 9)], sem_small),
        pltpu.async_copy(wpl_hbm, wv.at[pl.ds(32, 3)], sem_small),
        pltpu.async_copy(wn_hbm, wv.at[pl.ds(48, 7)], sem_small),
        pltpu.async_copy(wy_hbm, wv.at[pl.ds(64, 1)], sem_small),
        pltpu.async_copy(by_hbm, wv.at[pl.ds(80, 1)], sem_small),
    ]
    for c in small:
        c.wait()

    wpv = wv[pl.ds(0, _L)]
    wstv = wv[pl.ds(16, _L)]
    wplv = wv[pl.ds(32, _L)]
    wnv = wv[pl.ds(48, _L)]
    wyv = wv[pl.ds(64, _L)]
    byv = wv[pl.ds(80, _L)]

    lanes = lax.iota(jnp.int32, _L)

    pk = lanes * 2
    tP[...] = (plsc.load_gather(epv, [pk]) * wpv[0]
               + plsc.load_gather(epv, [pk + 1]) * wpv[1])
    for j in range(_SP // _L):
        sl = pl.ds(j * _L, _L)
        b = (lanes + (j * _L)) * 3
        e0 = plsc.load_gather(esv, [b])
        e1 = plsc.load_gather(esv, [b + 1])
        e2 = plsc.load_gather(esv, [b + 2])
        tSa[sl] = e0 * wstv[0] + e1 * wstv[1] + e2 * wstv[2]
        tSb[sl] = e0 * wstv[3] + e1 * wstv[4] + e2 * wstv[5]
        tSc[sl] = e0 * wstv[6] + e1 * wstv[7] + e2 * wstv[8]
    for j in range(_LP // _L):
        sl = pl.ds(j * _L, _L)
        b = (lanes + (j * _L)) * 3
        tL[sl] = (plsc.load_gather(elv, [b]) * wplv[0]
                  + plsc.load_gather(elv, [b + 1]) * wplv[1]
                  + plsc.load_gather(elv, [b + 2]) * wplv[2])

    for c in big:
        c.wait()

    for j in range(_CHUNKS):
        rbase = (lanes + (j * _L)) * 12
        sl = pl.ds(j * _L, _L)

        def col(c):
            return plsc.load_gather(xb, [rbase + c])

        acc = plsc.load_gather(tP, [col(0).astype(jnp.int32)])
        acc = acc + plsc.load_gather(tSa, [col(1).astype(jnp.int32)])
        acc = acc + plsc.load_gather(tSb, [col(2).astype(jnp.int32)])
        acc = acc + plsc.load_gather(tSc, [col(3).astype(jnp.int32)])
        acc = acc + plsc.load_gather(tL, [col(4).astype(jnp.int32)])
        for c in range(7):
            acc = acc + col(5 + c) * wnv[c]
        acc = acc + yv[sl] * wyv[0] + byv[0]
        ov[sl] = acc

    pltpu.sync_copy(ov, out_hbm.at[pl.ds(base, _RPW)])


@jax.jit
def _run(x, yf, ep, es, el, wp, wst, wpl, wn, wy, by):
    mesh = plsc.VectorSubcoreMesh(core_axis_name="c", subcore_axis_name="s",
                                  num_cores=_NC, num_subcores=_NS)
    f = pl.kernel(
        _sc_body,
        out_type=jax.ShapeDtypeStruct((_N,), jnp.float32),
        mesh=mesh,
        compiler_params=pltpu.CompilerParams(needs_layout_passes=False),
        scratch_types=[
            pltpu.VMEM((_RPW * 12,), jnp.float32),
            pltpu.VMEM((_RPW,), jnp.float32),
            pltpu.VMEM((_RPW,), jnp.float32),
            pltpu.VMEM((32,), jnp.float32),
            pltpu.VMEM((3 * _SP,), jnp.float32),
            pltpu.VMEM((3 * _LP,), jnp.float32),
            pltpu.VMEM((96,), jnp.float32),
            pltpu.VMEM((_L,), jnp.float32),
            pltpu.VMEM((_SP,), jnp.float32),
            pltpu.VMEM((_SP,), jnp.float32),
            pltpu.VMEM((_SP,), jnp.float32),
            pltpu.VMEM((_LP,), jnp.float32),
            pltpu.SemaphoreType.DMA,
            pltpu.SemaphoreType.DMA,
        ],
    )
    return f(x, yf, ep, es, el, wp, wst, wpl, wn, wy, by)


def kernel(x, y, emb_period, emb_stations, emb_platforms, w_period,
           w_stations, w_platforms, w_n, w_y, b_y):
    if x.ndim == 1:
        x = x.reshape(1, -1)
    out = _run(x.reshape(-1), y.reshape(-1),
               emb_period.reshape(-1), emb_stations.reshape(-1),
               emb_platforms.reshape(-1),
               w_period.reshape(-1), w_stations.reshape(-1),
               w_platforms.reshape(-1), w_n.reshape(-1),
               w_y.reshape(-1), b_y.reshape(-1))
    return out.reshape(-1, 1)


# speedup vs baseline: 8.8185x; 1.0487x over previous
import jax
import jax.numpy as jnp
from jax import lax
from jax.experimental import pallas as pl
from jax.experimental.pallas import tpu as pltpu
from jax.experimental.pallas import tpu_sc as plsc

_N = 16384
_NC = 2
_NS = 16
_NW = _NC * _NS
_RPW = _N // _NW
_L = 16
_CHUNKS = _RPW // _L

_SP = 176
_LP = 464


def _sc_body(x_hbm, y_hbm, ep_hbm, es_hbm, el_hbm, wp_hbm, wst_hbm,
             wpl_hbm, wn_hbm, wy_hbm, by_hbm, out_hbm,
             xb, yv, ov, epv, esv, elv, wv, tP, tSa, tSb, tSc, tL,
             sem_small, sem_big):
    wid = lax.axis_index("s") * _NC + lax.axis_index("c")
    base = wid * _RPW

    big = [
        pltpu.async_copy(x_hbm.at[pl.ds(base * 12, _RPW * 12)], xb, sem_big),
        pltpu.async_copy(y_hbm.at[pl.ds(base, _RPW)], yv, sem_big),
    ]
    small = [
        pltpu.async_copy(ep_hbm, epv.at[pl.ds(0, 8)], sem_small),
        pltpu.async_copy(es_hbm, esv.at[pl.ds(0, 486)], sem_small),
        pltpu.async_copy(el_hbm, elv.at[pl.ds(0, 1380)], sem_small),
        pltpu.async_copy(wp_hbm, wv.at[pl.ds(0, 2)], sem_small),
        pltpu.async_copy(wst_hbm, wv.at[pl.ds(16, 9)], sem_small),
        pltpu.async_copy(wpl_hbm, wv.at[pl.ds(32, 3)], sem_small),
        pltpu.async_copy(wn_hbm, wv.at[pl.ds(48, 7)], sem_small),
        pltpu.async_copy(wy_hbm, wv.at[pl.ds(64, 1)], sem_small),
        pltpu.async_copy(by_hbm, wv.at[pl.ds(80, 1)], sem_small),
    ]
    for c in small:
        c.wait()

    wpv = wv[pl.ds(0, _L)]
    wstv = wv[pl.ds(16, _L)]
    wplv = wv[pl.ds(32, _L)]
    wnv = wv[pl.ds(48, _L)]
    wyv = wv[pl.ds(64, _L)]
    byv = wv[pl.ds(80, _L)]

    lanes = lax.iota(jnp.int32, _L)

    pk = lanes * 2
    tP[...] = (plsc.load_gather(epv, [pk]) * wpv[0]
               + plsc.load_gather(epv, [pk + 1]) * wpv[1])

    @plsc.parallel_loop(0, _SP, step=_L)
    def _station_proj(i):
        sl = pl.ds(i, _L)
        b = (lanes + i) * 3
        e0 = plsc.load_gather(esv, [b])
        e1 = plsc.load_gather(esv, [b + 1])
        e2 = plsc.load_gather(esv, [b + 2])
        tSa[sl] = e0 * wstv[0] + e1 * wstv[1] + e2 * wstv[2]
        tSb[sl] = e0 * wstv[3] + e1 * wstv[4] + e2 * wstv[5]
        tSc[sl] = e0 * wstv[6] + e1 * wstv[7] + e2 * wstv[8]

    @plsc.parallel_loop(0, _LP, step=_L)
    def _platform_proj(i):
        b = (lanes + i) * 3
        tL[pl.ds(i, _L)] = (plsc.load_gather(elv, [b]) * wplv[0]
                            + plsc.load_gather(elv, [b + 1]) * wplv[1]
                            + plsc.load_gather(elv, [b + 2]) * wplv[2])

    for c in big:
        c.wait()

    @plsc.parallel_loop(0, _RPW, step=_L, unroll=2)
    def _rows(i):
        rbase = (lanes + i) * 12

        def col(c):
            return plsc.load_gather(xb, [rbase + c])

        acc = plsc.load_gather(tP, [col(0).astype(jnp.int32)])
        acc = acc + plsc.load_gather(tSa, [col(1).astype(jnp.int32)])
        acc = acc + plsc.load_gather(tSb, [col(2).astype(jnp.int32)])
        acc = acc + plsc.load_gather(tSc, [col(3).astype(jnp.int32)])
        acc = acc + plsc.load_gather(tL, [col(4).astype(jnp.int32)])
        for c in range(7):
            acc = acc + col(5 + c) * wnv[c]
        acc = acc + yv[pl.ds(i, _L)] * wyv[0] + byv[0]
        ov[pl.ds(i, _L)] = acc

    pltpu.sync_copy(ov, out_hbm.at[pl.ds(base, _RPW)])


@jax.jit
def _run(x, yf, ep, es, el, wp, wst, wpl, wn, wy, by):
    mesh = plsc.VectorSubcoreMesh(core_axis_name="c", subcore_axis_name="s",
                                  num_cores=_NC, num_subcores=_NS)
    f = pl.kernel(
        _sc_body,
        out_type=jax.ShapeDtypeStruct((_N,), jnp.float32),
        mesh=mesh,
        compiler_params=pltpu.CompilerParams(needs_layout_passes=False),
        scratch_types=[
            pltpu.VMEM((_RPW * 12,), jnp.float32),
            pltpu.VMEM((_RPW,), jnp.float32),
            pltpu.VMEM((_RPW,), jnp.float32),
            pltpu.VMEM((32,), jnp.float32),
            pltpu.VMEM((3 * _SP,), jnp.float32),
            pltpu.VMEM((3 * _LP,), jnp.float32),
            pltpu.VMEM((96,), jnp.float32),
            pltpu.VMEM((_L,), jnp.float32),
            pltpu.VMEM((_SP,), jnp.float32),
            pltpu.VMEM((_SP,), jnp.float32),
            pltpu.VMEM((_SP,), jnp.float32),
            pltpu.VMEM((_LP,), jnp.float32),
            pltpu.SemaphoreType.DMA,
            pltpu.SemaphoreType.DMA,
        ],
    )
    return f(x, yf, ep, es, el, wp, wst, wpl, wn, wy, by)


def kernel(x, y, emb_period, emb_stations, emb_platforms, w_period,
           w_stations, w_platforms, w_n, w_y, b_y):
    if x.ndim == 1:
        x = x.reshape(1, -1)
    out = _run(x.reshape(-1), y.reshape(-1),
               emb_period.reshape(-1), emb_stations.reshape(-1),
               emb_platforms.reshape(-1),
               w_period.reshape(-1), w_stations.reshape(-1),
               w_platforms.reshape(-1), w_n.reshape(-1),
               w_y.reshape(-1), b_y.reshape(-1))
    return out.reshape(-1, 1)


# speedup vs baseline: 8.8499x; 1.0036x over previous
import jax
import jax.numpy as jnp
from jax import lax
from jax.experimental import pallas as pl
from jax.experimental.pallas import tpu as pltpu
from jax.experimental.pallas import tpu_sc as plsc

_N = 16384
_NC = 2
_NS = 16
_NW = _NC * _NS
_RPW = _N // _NW
_L = 16
_CHUNKS = _RPW // _L

_SP = 176
_LP = 464


def _sc_body(x_hbm, y_hbm, ep_hbm, es_hbm, el_hbm, wp_hbm, wst_hbm,
             wpl_hbm, wn_hbm, wy_hbm, by_hbm, out_hbm,
             xb, yv, ov, epv, esv, elv, wv, tP, tSa, tSb, tSc, tL,
             sem_small, sem_big):
    wid = lax.axis_index("s") * _NC + lax.axis_index("c")
    base = wid * _RPW

    big = [
        pltpu.async_copy(x_hbm.at[pl.ds(base * 12, _RPW * 12)], xb, sem_big),
        pltpu.async_copy(y_hbm.at[pl.ds(base, _RPW)], yv, sem_big),
    ]
    small = [
        pltpu.async_copy(ep_hbm, epv.at[pl.ds(0, 8)], sem_small),
        pltpu.async_copy(es_hbm, esv.at[pl.ds(0, 486)], sem_small),
        pltpu.async_copy(el_hbm, elv.at[pl.ds(0, 1380)], sem_small),
        pltpu.async_copy(wp_hbm, wv.at[pl.ds(0, 2)], sem_small),
        pltpu.async_copy(wst_hbm, wv.at[pl.ds(16, 9)], sem_small),
        pltpu.async_copy(wpl_hbm, wv.at[pl.ds(32, 3)], sem_small),
        pltpu.async_copy(wn_hbm, wv.at[pl.ds(48, 7)], sem_small),
        pltpu.async_copy(wy_hbm, wv.at[pl.ds(64, 1)], sem_small),
        pltpu.async_copy(by_hbm, wv.at[pl.ds(80, 1)], sem_small),
    ]
    for c in small:
        c.wait()

    wpv = wv[pl.ds(0, _L)]
    wstv = wv[pl.ds(16, _L)]
    wplv = wv[pl.ds(32, _L)]
    wnv = wv[pl.ds(48, _L)]
    wyv = wv[pl.ds(64, _L)]
    byv = wv[pl.ds(80, _L)]

    lanes = lax.iota(jnp.int32, _L)

    pk = lanes * 2
    tP[...] = (plsc.load_gather(epv, [pk]) * wpv[0]
               + plsc.load_gather(epv, [pk + 1]) * wpv[1])

    @plsc.parallel_loop(0, _SP, step=_L)
    def _station_proj(i):
        sl = pl.ds(i, _L)
        b = (lanes + i) * 3
        e0 = plsc.load_gather(esv, [b])
        e1 = plsc.load_gather(esv, [b + 1])
        e2 = plsc.load_gather(esv, [b + 2])
        tSa[sl] = e0 * wstv[0] + e1 * wstv[1] + e2 * wstv[2]
        tSb[sl] = e0 * wstv[3] + e1 * wstv[4] + e2 * wstv[5]
        tSc[sl] = e0 * wstv[6] + e1 * wstv[7] + e2 * wstv[8]

    @plsc.parallel_loop(0, _LP, step=_L)
    def _platform_proj(i):
        b = (lanes + i) * 3
        tL[pl.ds(i, _L)] = (plsc.load_gather(elv, [b]) * wplv[0]
                            + plsc.load_gather(elv, [b + 1]) * wplv[1]
                            + plsc.load_gather(elv, [b + 2]) * wplv[2])

    for c in big:
        c.wait()

    @plsc.parallel_loop(0, _RPW, step=_L, unroll=2)
    def _rows(i):
        rbase = (lanes + i) * 12

        def col(c):
            return plsc.load_gather(xb, [rbase + c])

        acc = plsc.load_gather(tP, [col(0).astype(jnp.int32)])
        acc = acc + plsc.load_gather(tSa, [col(1).astype(jnp.int32)])
        acc = acc + plsc.load_gather(tSb, [col(2).astype(jnp.int32)])
        acc = acc + plsc.load_gather(tSc, [col(3).astype(jnp.int32)])
        acc = acc + plsc.load_gather(tL, [col(4).astype(jnp.int32)])
        for c in range(7):
            acc = acc + col(5 + c) * wnv[c]
        acc = acc + yv[pl.ds(i, _L)] * wyv[0] + byv[0]
        ov[pl.ds(i, _L)] = acc

    pltpu.sync_copy(ov, out_hbm.at[pl.ds(base, _RPW)])


@jax.jit
def _run(x, yf, ep, es, el, wp, wst, wpl, wn, wy, by):
    mesh = plsc.VectorSubcoreMesh(core_axis_name="c", subcore_axis_name="s",
                                  num_cores=_NC, num_subcores=_NS)
    f = pl.kernel(
        _sc_body,
        out_type=jax.ShapeDtypeStruct((_N,), jnp.float32),
        mesh=mesh,
        compiler_params=pltpu.CompilerParams(needs_layout_passes=False,
                                             skip_device_barrier=True),
        scratch_types=[
            pltpu.VMEM((_RPW * 12,), jnp.float32),
            pltpu.VMEM((_RPW,), jnp.float32),
            pltpu.VMEM((_RPW,), jnp.float32),
            pltpu.VMEM((32,), jnp.float32),
            pltpu.VMEM((3 * _SP,), jnp.float32),
            pltpu.VMEM((3 * _LP,), jnp.float32),
            pltpu.VMEM((96,), jnp.float32),
            pltpu.VMEM((_L,), jnp.float32),
            pltpu.VMEM((_SP,), jnp.float32),
            pltpu.VMEM((_SP,), jnp.float32),
            pltpu.VMEM((_SP,), jnp.float32),
            pltpu.VMEM((_LP,), jnp.float32),
            pltpu.SemaphoreType.DMA,
            pltpu.SemaphoreType.DMA,
        ],
    )
    return f(x, yf, ep, es, el, wp, wst, wpl, wn, wy, by)


def kernel(x, y, emb_period, emb_stations, emb_platforms, w_period,
           w_stations, w_platforms, w_n, w_y, b_y):
    if x.ndim == 1:
        x = x.reshape(1, -1)
    out = _run(x.reshape(-1), y.reshape(-1),
               emb_period.reshape(-1), emb_stations.reshape(-1),
               emb_platforms.reshape(-1),
               w_period.reshape(-1), w_stations.reshape(-1),
               w_platforms.reshape(-1), w_n.reshape(-1),
               w_y.reshape(-1), b_y.reshape(-1))
    return out.reshape(-1, 1)
